# baseline (speedup 1.0000x reference)
import functools

import jax
import jax.numpy as jnp
from jax import lax
from jax.experimental import pallas as pl
from jax.experimental.pallas import tpu as pltpu
from jax.experimental.pallas import tpu_sc as plsc

_N = 10000
_E = 160000
_D = 256
_OUT = 256
_HALF = 128
_KS = 3

_NCORE = 2
_NSUB = 16
_ESUB = _E // _NSUB
_B = 80
_NB = _ESUB // _B
_ROWS_T = 624
_ROWS_REM = _N - _NSUB * _ROWS_T
_LANES = 16


_NSLOT = 4


def _spmm_kernel(xflat, rowi, coli, vals, ycat, *scr):
    dst_s = scr[0:4]
    src_s = scr[4:8]
    vals_s = scr[8:12]
    dsts_s = scr[12:16]
    rows_s = scr[16:20]
    acc = scr[20]
    sem_i = scr[21:25]
    sem_g = scr[25:29]
    sem_s = scr[29:33]
    cid = lax.axis_index("c")
    sid = lax.axis_index("s")

    zero16 = jnp.zeros((_LANES,), jnp.float32)

    def zrow(i, carry):
        for j in range(_HALF // _LANES):
            rows_s[0][i, pl.ds(j * _LANES, _LANES)] = zero16
        return carry

    lax.fori_loop(0, _B, zrow, 0)

    zbase = sid * _ROWS_T
    for k in range(_ROWS_T // _B):
        pltpu.sync_copy(rows_s[0], acc.at[pl.ds(zbase + k * _B, _B)])
    pltpu.sync_copy(rows_s[0].at[pl.ds(0, _ROWS_T % _B)],
                    acc.at[pl.ds(zbase + (_ROWS_T // _B) * _B, _ROWS_T % _B)])

    @pl.when(sid == 0)
    def _zero_tail():
        pltpu.sync_copy(rows_s[0].at[pl.ds(0, _ROWS_REM)],
                        acc.at[pl.ds(_NSUB * _ROWS_T, _ROWS_REM)])

    plsc.subcore_barrier()

    ebase = sid * _ESUB
    col_off = cid

    def start_idx(b, s):
        base = ebase + b * _B
        pltpu.async_copy(rowi.at[pl.ds(base, _B)], dst_s[s], sem_i[s])
        pltpu.async_copy(coli.at[pl.ds(base, _B)], src_s[s], sem_i[s])
        pltpu.async_copy(vals.at[pl.ds(base, _B)], vals_s[s], sem_i[s])

    def wait_idx(b, s):
        base = ebase + b * _B
        pltpu.make_async_copy(rowi.at[pl.ds(base, _B)], dst_s[s],
                              sem_i[s]).wait()
        pltpu.make_async_copy(coli.at[pl.ds(base, _B)], src_s[s],
                              sem_i[s]).wait()
        pltpu.make_async_copy(vals.at[pl.ds(base, _B)], vals_s[s],
                              sem_i[s]).wait()
        for j in range(_B // _LANES):
            sl = pl.ds(j * _LANES, _LANES)
            src_s[s][sl] = src_s[s][sl] + src_s[s][sl] + col_off

    def start_gather(s):
        pltpu.async_copy(xflat.at[src_s[s]], rows_s[s], sem_g[s])

    def wait_gather(s):
        pltpu.make_async_copy(xflat.at[src_s[s]], rows_s[s], sem_g[s]).wait()

    def scale(s):
        def edge(i0, ecarry):
            for k in range(8):
                e = 8 * i0 + k
                vv = plsc.load_gather(
                    vals_s[s], [jnp.full((_LANES,), e, jnp.int32)])
                for j in range(_HALF // _LANES):
                    rows_s[s][e, pl.ds(j * _LANES, _LANES)] = (
                        rows_s[s][e, pl.ds(j * _LANES, _LANES)] * vv)
            return ecarry

        lax.fori_loop(0, _B // 8, edge, 0)

    def start_scatter(s):
        for j in range(_B // _LANES):
            sl = pl.ds(j * _LANES, _LANES)
            dsts_s[s][sl] = dst_s[s][sl]
        pltpu.async_copy(rows_s[s], acc.at[dsts_s[s]], sem_s[s], add=True)

    def wait_scatter(s):
        pltpu.make_async_copy(rows_s[s], acc.at[dsts_s[s]],
                              sem_s[s]).wait()

    for t in range(_NSLOT):
        start_idx(t, t)
    for t in range(2):
        wait_idx(t, t)
        start_gather(t)

    def ring(i, carry):
        g = 4 * i
        for k in range(_NSLOT):
            b = g + k
            wait_gather(k)
            scale(k)
            start_scatter(k)

            @pl.when(b + 4 < _NB)
            def _prefetch_idx(b=b, k=k):
                start_idx(b + 4, k)

            k2 = (k + 2) % _NSLOT

            @pl.when(b + 2 < _NB)
            def _advance(b=b, k2=k2):
                @pl.when(b >= 2)
                def _free_slot():
                    wait_scatter(k2)

                wait_idx(b + 2, k2)
                start_gather(k2)
        return carry

    lax.fori_loop(0, (_NB - 1) // _NSLOT, ring, 0)

    wait_gather(0)
    scale(0)
    start_scatter(0)
    for t in range(_NSLOT):
        wait_scatter((_NB - 4 + t) % _NSLOT)
    plsc.subcore_barrier()

    pltpu.sync_copy(
        acc.at[pl.ds(sid * _ROWS_T, _ROWS_T)],
        ycat.at[pl.ds(sid * _ROWS_T, _ROWS_T), cid])

    @pl.when(sid == 0)
    def _copy_tail():
        pltpu.sync_copy(
            acc.at[pl.ds(_NSUB * _ROWS_T, _ROWS_REM)],
            ycat.at[pl.ds(_NSUB * _ROWS_T, _ROWS_REM), cid])


_SPMM = functools.partial(
    pl.kernel,
    out_type=jax.ShapeDtypeStruct((_N, _NCORE, _HALF), jnp.float32),
    mesh=plsc.VectorSubcoreMesh(core_axis_name="c", subcore_axis_name="s"),
    scratch_types=(
        [pltpu.VMEM((_B,), jnp.int32) for _ in range(_NSLOT)]
        + [pltpu.VMEM((_B,), jnp.int32) for _ in range(_NSLOT)]
        + [pltpu.VMEM((_B,), jnp.float32) for _ in range(_NSLOT)]
        + [pltpu.VMEM((_B,), jnp.int32) for _ in range(_NSLOT)]
        + [pltpu.VMEM((_B, _HALF), jnp.float32) for _ in range(_NSLOT)]
        + [pltpu.VMEM_SHARED((_N, _HALF), jnp.float32)]
        + [pltpu.SemaphoreType.DMA for _ in range(3 * _NSLOT)]
    ),
    compiler_params=pltpu.CompilerParams(needs_layout_passes=False),
)(_spmm_kernel)


_BM = 1000


def _mm_body(x_ref, y_ref, z_ref, w_ref, b_ref, o_ref):
    xk = jnp.concatenate([x_ref[...], y_ref[...], z_ref[...]], axis=1)
    o_ref[...] = jnp.dot(xk, w_ref[...],
                         preferred_element_type=jnp.float32) + b_ref[...]


def _matmul(x, y, z, wc, bias):
    dspec = pl.BlockSpec((_BM, _D), lambda i: (i, 0))
    return pl.pallas_call(
        _mm_body,
        grid=(_N // _BM,),
        in_specs=[
            dspec, dspec, dspec,
            pl.BlockSpec((_D * _KS, _OUT), lambda i: (0, 0)),
            pl.BlockSpec((1, _OUT), lambda i: (0, 0)),
        ],
        out_specs=pl.BlockSpec((_BM, _OUT), lambda i: (i, 0)),
        out_shape=jax.ShapeDtypeStruct((_N, _OUT), jnp.float32),
    )(x, y, z, wc, bias)


def kernel(x, support_indices, support_values, weight, biases):
    rowi = support_indices[0]
    coli = support_indices[1]
    xflat = x.reshape(_NCORE * _N, _HALF)
    y = _SPMM(xflat, rowi, coli, support_values)
    z = _SPMM(y.reshape(_NCORE * _N, _HALF), rowi, coli, support_values)

    wr = weight.reshape(_D, _KS, _OUT)
    w0, w1, w2 = wr[:, 0], wr[:, 1], wr[:, 2]
    wc = jnp.concatenate([w0 - w2, w1, 2.0 * w2], axis=0)
    return _matmul(x, y.reshape(_N, _D), z.reshape(_N, _D), wc,
                   biases.reshape(1, _OUT))


# speedup vs baseline: 1.0202x; 1.0202x over previous
import functools

import jax
import jax.numpy as jnp
from jax import lax
from jax.experimental import pallas as pl
from jax.experimental.pallas import tpu as pltpu
from jax.experimental.pallas import tpu_sc as plsc

_N = 10000
_E = 160000
_D = 256
_OUT = 256
_HALF = 128
_KS = 3

_NCORE = 2
_NSUB = 16
_ESUB = _E // _NSUB
_B = 80
_NB = _ESUB // _B
_ROWS_T = 624
_ROWS_REM = _N - _NSUB * _ROWS_T
_LANES = 16


_NSLOT = 4


def _spmm_kernel(xflat, rowi, coli, vals, yout, zout, *scr):
    dst_s = scr[0:4]
    src_s = scr[4:8]
    vals_s = scr[8:12]
    dsts_s = scr[12:16]
    rows_s = scr[16:20]
    acc = scr[20]
    sem_i = scr[21:25]
    sem_g = scr[25:29]
    sem_s = scr[29:33]
    cid = lax.axis_index("c")
    sid = lax.axis_index("s")

    ebase = sid * _ESUB
    zero16 = jnp.zeros((_LANES,), jnp.float32)

    def start_idx(b, s):
        base = ebase + b * _B
        pltpu.async_copy(rowi.at[pl.ds(base, _B)], dst_s[s], sem_i[s])
        pltpu.async_copy(coli.at[pl.ds(base, _B)], src_s[s], sem_i[s])
        pltpu.async_copy(vals.at[pl.ds(base, _B)], vals_s[s], sem_i[s])

    col_off = cid * _N

    def wait_idx(b, s):
        base = ebase + b * _B
        pltpu.make_async_copy(rowi.at[pl.ds(base, _B)], dst_s[s],
                              sem_i[s]).wait()
        pltpu.make_async_copy(coli.at[pl.ds(base, _B)], src_s[s],
                              sem_i[s]).wait()
        pltpu.make_async_copy(vals.at[pl.ds(base, _B)], vals_s[s],
                              sem_i[s]).wait()
        for j in range(_B // _LANES):
            sl = pl.ds(j * _LANES, _LANES)
            src_s[s][sl] = src_s[s][sl] + col_off

    def scale(s):
        def edge(i0, ecarry):
            for k in range(8):
                e = 8 * i0 + k
                vv = plsc.load_gather(
                    vals_s[s], [jnp.full((_LANES,), e, jnp.int32)])
                for j in range(_HALF // _LANES):
                    rows_s[s][e, pl.ds(j * _LANES, _LANES)] = (
                        rows_s[s][e, pl.ds(j * _LANES, _LANES)] * vv)
            return ecarry

        lax.fori_loop(0, _B // 8, edge, 0)

    def start_scatter(s):
        for j in range(_B // _LANES):
            sl = pl.ds(j * _LANES, _LANES)
            dsts_s[s][sl] = dst_s[s][sl]
        pltpu.async_copy(rows_s[s], acc.at[dsts_s[s]], sem_s[s], add=True)

    def wait_scatter(s):
        pltpu.make_async_copy(rows_s[s], acc.at[dsts_s[s]],
                              sem_s[s]).wait()

    def spmm_round(table, out):
        def zrow(i, carry):
            for j in range(_HALF // _LANES):
                rows_s[0][i, pl.ds(j * _LANES, _LANES)] = zero16
            return carry

        lax.fori_loop(0, _B, zrow, 0)

        zbase = sid * _ROWS_T
        for k in range(_ROWS_T // _B):
            pltpu.sync_copy(rows_s[0], acc.at[pl.ds(zbase + k * _B, _B)])
        pltpu.sync_copy(
            rows_s[0].at[pl.ds(0, _ROWS_T % _B)],
            acc.at[pl.ds(zbase + (_ROWS_T // _B) * _B, _ROWS_T % _B)])

        @pl.when(sid == 0)
        def _zero_tail():
            pltpu.sync_copy(rows_s[0].at[pl.ds(0, _ROWS_REM)],
                            acc.at[pl.ds(_NSUB * _ROWS_T, _ROWS_REM)])

        plsc.subcore_barrier()

        def start_gather(s):
            pltpu.async_copy(table.at[src_s[s]], rows_s[s], sem_g[s])

        def wait_gather(s):
            pltpu.make_async_copy(table.at[src_s[s]], rows_s[s],
                                  sem_g[s]).wait()

        for t in range(_NSLOT):
            start_idx(t, t)
        for t in range(2):
            wait_idx(t, t)
            start_gather(t)

        def ring(i, carry):
            g = 4 * i
            for k in range(_NSLOT):
                b = g + k
                wait_gather(k)
                scale(k)
                start_scatter(k)

                @pl.when(b + 4 < _NB)
                def _prefetch_idx(b=b, k=k):
                    start_idx(b + 4, k)

                k2 = (k + 2) % _NSLOT

                @pl.when(b + 2 < _NB)
                def _advance(b=b, k2=k2):
                    @pl.when(b >= 2)
                    def _free_slot():
                        wait_scatter(k2)

                    wait_idx(b + 2, k2)
                    start_gather(k2)
            return carry

        lax.fori_loop(0, (_NB - 1) // _NSLOT, ring, 0)

        wait_gather(0)
        scale(0)
        start_scatter(0)
        for t in range(_NSLOT):
            wait_scatter((_NB - 4 + t) % _NSLOT)
        plsc.subcore_barrier()

        pltpu.sync_copy(
            acc.at[pl.ds(sid * _ROWS_T, _ROWS_T)],
            out.at[pl.ds(col_off + sid * _ROWS_T, _ROWS_T)])

        @pl.when(sid == 0)
        def _copy_tail():
            pltpu.sync_copy(
                acc.at[pl.ds(_NSUB * _ROWS_T, _ROWS_REM)],
                out.at[pl.ds(col_off + _NSUB * _ROWS_T, _ROWS_REM)])

        plsc.subcore_barrier()

    spmm_round(xflat, yout)
    spmm_round(yout, zout)


_SPMM = functools.partial(
    pl.kernel,
    out_type=(jax.ShapeDtypeStruct((_NCORE * _N, _HALF), jnp.float32),
              jax.ShapeDtypeStruct((_NCORE * _N, _HALF), jnp.float32)),
    mesh=plsc.VectorSubcoreMesh(core_axis_name="c", subcore_axis_name="s"),
    scratch_types=(
        [pltpu.VMEM((_B,), jnp.int32) for _ in range(_NSLOT)]
        + [pltpu.VMEM((_B,), jnp.int32) for _ in range(_NSLOT)]
        + [pltpu.VMEM((_B,), jnp.float32) for _ in range(_NSLOT)]
        + [pltpu.VMEM((_B,), jnp.int32) for _ in range(_NSLOT)]
        + [pltpu.VMEM((_B, _HALF), jnp.float32) for _ in range(_NSLOT)]
        + [pltpu.VMEM_SHARED((_N, _HALF), jnp.float32)]
        + [pltpu.SemaphoreType.DMA for _ in range(3 * _NSLOT)]
    ),
    compiler_params=pltpu.CompilerParams(needs_layout_passes=False),
)(_spmm_kernel)


_BM = 1000


def _mm_body(x_ref, y0_ref, y1_ref, z0_ref, z1_ref, w_ref, b_ref, o_ref):
    xk = jnp.concatenate(
        [x_ref[...], y0_ref[...], y1_ref[...], z0_ref[...], z1_ref[...]],
        axis=1)
    o_ref[...] = jnp.dot(xk, w_ref[...],
                         preferred_element_type=jnp.float32) + b_ref[...]


def _matmul(x, y, z, wc, bias):
    hspec = pl.BlockSpec((_BM, _HALF), lambda i: (i, 0))
    return pl.pallas_call(
        _mm_body,
        grid=(_N // _BM,),
        in_specs=[
            pl.BlockSpec((_BM, _D), lambda i: (i, 0)),
            hspec, hspec, hspec, hspec,
            pl.BlockSpec((_D * _KS, _OUT), lambda i: (0, 0)),
            pl.BlockSpec((1, _OUT), lambda i: (0, 0)),
        ],
        out_specs=pl.BlockSpec((_BM, _OUT), lambda i: (i, 0)),
        out_shape=jax.ShapeDtypeStruct((_N, _OUT), jnp.float32),
    )(x, y[:_N], y[_N:], z[:_N], z[_N:], wc, bias)


def kernel(x, support_indices, support_values, weight, biases):
    rowi = support_indices[0]
    coli = support_indices[1]
    xflat = jnp.concatenate([x[:, :_HALF], x[:, _HALF:]], axis=0)
    y, z = _SPMM(xflat, rowi, coli, support_values)

    wr = weight.reshape(_D, _KS, _OUT)
    w0, w1, w2 = wr[:, 0], wr[:, 1], wr[:, 2]
    wc = jnp.concatenate(
        [w0 - w2, w1[:_HALF], w1[_HALF:], 2.0 * w2[:_HALF], 2.0 * w2[_HALF:]],
        axis=0)
    return _matmul(x, y, z, wc, biases.reshape(1, _OUT))
